# baseline (speedup 1.0000x reference)
import functools

import jax
import jax.numpy as jnp
from jax import lax
from jax.experimental import pallas as pl
from jax.experimental.pallas import tpu as pltpu
from jax.experimental.pallas import tpu_sc as plsc


def _proj_kernel(x_ref, w_ref, o_ref):
    o_ref[...] = jnp.dot(x_ref[...], w_ref[...],
                         preferred_element_type=jnp.float32)


def _make_select(N, CH, BLK):
    def body(c0_ref, nact_ref, q_ref, pt_ref, xs_ref, w2_ref, b_ref,
             part_ref, idx_ref, wex_ref):
        g = pl.program_id(0)
        a = q_ref[...]
        q0 = a[:, 0:1]
        q1 = a[:, 1:2]
        q2 = a[:, 2:3]
        bq = a[:, 3:4]
        qn = q0 * q0 + q1 * q1 + q2 * q2

        def _t(v):
            return v.astype(jnp.bfloat16).astype(jnp.float32)
        r0 = _t(q0)
        r1 = _t(q1)
        r2 = _t(q2)

        def chunk_body(c, carry):
            base = c * CH
            pt = pt_ref[:, pl.ds(base, CH)]
            p0 = pt[0:1, :]
            p1 = pt[1:2, :]
            p2 = pt[2:3, :]
            bp = pt[3:4, :]
            s0 = _t(p0)
            s1 = _t(p1)
            s2 = _t(p2)
            pn = p0 * p0 + p1 * p1 + p2 * p2
            cross = r0 * s0 + r1 * s1 + r2 * s2
            d2 = jnp.maximum((qn + pn) - 2.0 * cross, 0.0)
            d2 = jnp.where(bq != bp, 1e10, d2)

            colg = lax.broadcasted_iota(jnp.int32, d2.shape, 1) + base
            loc = []
            for _ in range(3):
                m = jnp.min(d2, axis=1, keepdims=True)
                cand = jnp.where(d2 == m, colg, 2 * N)
                j = jnp.min(cand, axis=1, keepdims=True)
                loc.append((m, j))
                d2 = jnp.where(cand == j, 3e10, d2)

            A = [(carry[k], carry[3 + k]) for k in range(3)]
            B = loc

            def lexlt(x, y):
                return (x[0] < y[0]) | ((x[0] == y[0]) & (x[1] < y[1]))

            def lmin(x, y):
                lt = lexlt(x, y)
                return (jnp.where(lt, x[0], y[0]), jnp.where(lt, x[1], y[1]))

            def lmax(x, y):
                lt = lexlt(x, y)
                return (jnp.where(lt, y[0], x[0]), jnp.where(lt, y[1], x[1]))

            m1 = lmin(A[0], B[0])
            m2 = lmin(lmin(lmax(A[0], B[0]), A[1]), B[1])
            m3 = lmin(lmin(A[2], lmax(A[1], B[0])),
                      lmin(lmax(A[0], B[1]), B[2]))
            return (m1[0], m2[0], m3[0], m1[1], m2[1], m3[1])

        init = (jnp.full((BLK, 1), 4e10, jnp.float32),) * 3 \
            + (jnp.full((BLK, 1), N, jnp.int32),) * 3
        c0 = c0_ref[g]
        res = lax.fori_loop(c0, c0 + nact_ref[g], chunk_body, init)

        part_ref[...] = (jnp.dot(xs_ref[...], w2_ref[...],
                                 preferred_element_type=jnp.float32)
                         + b_ref[0:1, :])
        ws = []
        den = jnp.zeros((BLK, 1), jnp.float32)
        for k in range(3):
            wk = 1.0 / jnp.maximum(res[k], 1e-16)
            ws.append(wk)
            den = den + wk
        inv_den = 1.0 / den
        idx_ref[...] = jnp.concatenate([res[3], res[4], res[5]], axis=1)
        wex_ref[...] = jnp.concatenate(
            [jnp.broadcast_to(w * inv_den, (BLK, 16)) for w in ws], axis=1)

    return body


_SC_CHUNK = 32


def _sc_combine(xp_hbm, idxf_hbm, wex_hbm, part_hbm, out_hbm,
                idx_v, w_v, g_v, o_v, in_sem, out_sem):
    nw = 32
    m_rows = out_hbm.shape[0]
    rows_per_w = m_rows // nw
    wid = lax.axis_index("s") * 2 + lax.axis_index("c")
    base = wid * rows_per_w
    chunk = _SC_CHUNK
    nchunks = rows_per_w // chunk
    fcols = out_hbm.shape[1]
    nvec = fcols // 16

    def fire(ci, slot):
        rbase = base + ci * chunk
        pltpu.sync_copy(idxf_hbm.at[pl.ds(rbase * 3, chunk * 3)], idx_v[slot])
        g = pltpu.async_copy(xp_hbm.at[idx_v[slot]], g_v[slot], in_sem[slot])
        w = pltpu.async_copy(wex_hbm.at[pl.ds(rbase, chunk)], w_v[slot],
                             in_sem[slot])
        p = pltpu.async_copy(part_hbm.at[pl.ds(rbase, chunk)], o_v[slot],
                             in_sem[slot])
        return (g, w, p)

    pending = {0: fire(0, 0)}
    out_pending = [None, None]

    for ci in range(nchunks):
        slot = ci % 2
        if ci + 1 < nchunks:
            nxt = (ci + 1) % 2
            if out_pending[nxt] is not None:
                out_pending[nxt].wait()
                out_pending[nxt] = None
            pending[nxt] = fire(ci + 1, nxt)
        for d in pending[slot]:
            d.wait()

        def row_body(r, _, slot=slot):
            w0 = w_v[slot][r, 0:16]
            w1 = w_v[slot][r, 16:32]
            w2 = w_v[slot][r, 32:48]
            for c in range(nvec):
                sl = pl.ds(c * 16, 16)
                acc = (g_v[slot][3 * r, sl] * w0 + g_v[slot][3 * r + 1, sl] * w1
                       + g_v[slot][3 * r + 2, sl] * w2 + o_v[slot][r, sl])
                o_v[slot][r, sl] = acc
            return 0

        lax.fori_loop(0, chunk, row_body, 0)
        rbase = base + ci * chunk
        out_pending[slot] = pltpu.async_copy(
            o_v[slot], out_hbm.at[pl.ds(rbase, chunk)], out_sem[slot])

    for s in (0, 1):
        if out_pending[s] is not None:
            out_pending[s].wait()


def kernel(x, pos, batch, x_skip, pos_skip, batch_skip, W, b):
    M = pos_skip.shape[0]
    N = pos.shape[0]
    F = x.shape[1]
    Fs = x_skip.shape[1]
    BLK = 256

    W1 = W[:F, :]
    W2 = W[F:, :]

    xp = pl.pallas_call(
        _proj_kernel,
        grid=(8,),
        in_specs=[pl.BlockSpec((N // 8, F), lambda i: (i, 0)),
                  pl.BlockSpec((F, F), lambda i: (0, 0))],
        out_specs=pl.BlockSpec((N // 8, F), lambda i: (i, 0)),
        out_shape=jax.ShapeDtypeStruct((N, F), jnp.float32),
    )(x, W1)

    q = jnp.concatenate(
        [pos_skip, batch_skip.astype(jnp.float32)[:, None],
         jnp.zeros((M, 4), jnp.float32)], axis=1)
    pt_arr = jnp.concatenate(
        [pos.T, batch.astype(jnp.float32)[None, :],
         jnp.zeros((4, N), jnp.float32)], axis=0)
    b_arr = jnp.zeros((8, F), jnp.float32).at[0].set(b)

    CH = 512
    nchunks = N // CH
    seg_start = jnp.searchsorted(batch, jnp.arange(16, dtype=batch.dtype),
                                 side="left")
    seg_end = jnp.searchsorted(batch, jnp.arange(16, dtype=batch.dtype),
                               side="right")
    bmin = batch_skip[::BLK]
    bmax = batch_skip[BLK - 1::BLK]
    wstart = seg_start[bmin]
    wend = seg_end[bmax]
    c0 = (wstart // CH).astype(jnp.int32)
    c1 = jnp.maximum((jnp.maximum(wend, 1) - 1) // CH, c0)
    nact = (c1 - c0 + 1).astype(jnp.int32)

    halves = 2
    rows_h = M // halves
    blocks_h = rows_h // BLK
    outs = []
    for h in range(halves):
        r0 = h * rows_h
        part, idx3, wex = pl.pallas_call(
            _make_select(N, CH, BLK),
            grid_spec=pltpu.PrefetchScalarGridSpec(
                num_scalar_prefetch=2,
                grid=(blocks_h,),
                in_specs=[
                    pl.BlockSpec((BLK, 8), lambda g, *_: (g, 0)),
                    pl.BlockSpec((8, N), lambda g, *_: (0, 0)),
                    pl.BlockSpec((BLK, Fs), lambda g, *_: (g, 0)),
                    pl.BlockSpec((Fs, F), lambda g, *_: (0, 0)),
                    pl.BlockSpec((8, F), lambda g, *_: (0, 0)),
                ],
                out_specs=[
                    pl.BlockSpec((BLK, F), lambda g, *_: (g, 0)),
                    pl.BlockSpec((BLK, 3), lambda g, *_: (g, 0)),
                    pl.BlockSpec((BLK, 48), lambda g, *_: (g, 0)),
                ],
            ),
            out_shape=[
                jax.ShapeDtypeStruct((rows_h, F), jnp.float32),
                jax.ShapeDtypeStruct((rows_h, 3), jnp.int32),
                jax.ShapeDtypeStruct((rows_h, 48), jnp.float32),
            ],
        )(c0[h * blocks_h:(h + 1) * blocks_h],
          nact[h * blocks_h:(h + 1) * blocks_h],
          q[r0:r0 + rows_h], pt_arr, x_skip[r0:r0 + rows_h], W2, b_arr)
        outs.append(_sc_call(xp, idx3.reshape(rows_h * 3), wex, part,
                             rows_h, F))

    out = jnp.concatenate(outs, axis=0)

    return (out, pos_skip, batch_skip)


def _sc_call(xp, idx_flat, wex, part, M, F):
    chunk = _SC_CHUNK
    sc_fn = functools.partial(
        pl.kernel,
        out_type=jax.ShapeDtypeStruct((M, F), jnp.float32),
        mesh=plsc.VectorSubcoreMesh(core_axis_name="c", subcore_axis_name="s"),
        scratch_types=[
            [pltpu.VMEM((chunk * 3,), jnp.int32)] * 2,
            [pltpu.VMEM((chunk, 48), jnp.float32)] * 2,
            [pltpu.VMEM((chunk * 3, F), jnp.float32)] * 2,
            [pltpu.VMEM((chunk, F), jnp.float32)] * 2,
            [pltpu.SemaphoreType.DMA] * 2,
            [pltpu.SemaphoreType.DMA] * 2,
        ],
    )(_sc_combine)
    return sc_fn(xp, idx_flat, wex, part)


# speedup vs baseline: 1.0056x; 1.0056x over previous
import functools

import jax
import jax.numpy as jnp
from jax import lax
from jax.experimental import pallas as pl
from jax.experimental.pallas import tpu as pltpu
from jax.experimental.pallas import tpu_sc as plsc


def _proj_kernel(x_ref, w_ref, o_ref):
    o_ref[...] = jnp.dot(x_ref[...], w_ref[...],
                         preferred_element_type=jnp.float32)


def _make_select(N, CH, BLK):
    def body(c0_ref, nact_ref, q_ref, pt_ref, xs_ref, w2_ref, b_ref,
             part_ref, idx_ref, wex_ref):
        g = pl.program_id(0)
        a = q_ref[...]
        q0 = a[:, 0:1]
        q1 = a[:, 1:2]
        q2 = a[:, 2:3]
        bq = a[:, 3:4]
        qn = q0 * q0 + q1 * q1 + q2 * q2

        def _t(v):
            return v.astype(jnp.bfloat16).astype(jnp.float32)
        r0 = _t(q0)
        r1 = _t(q1)
        r2 = _t(q2)

        def chunk_body(c, carry):
            base = c * CH
            pt = pt_ref[:, pl.ds(base, CH)]
            p0 = pt[0:1, :]
            p1 = pt[1:2, :]
            p2 = pt[2:3, :]
            bp = pt[3:4, :]
            s0 = _t(p0)
            s1 = _t(p1)
            s2 = _t(p2)
            pn = p0 * p0 + p1 * p1 + p2 * p2
            cross = r0 * s0 + r1 * s1 + r2 * s2
            d2 = jnp.maximum((qn + pn) - 2.0 * cross, 0.0)
            d2 = jnp.where(bq != bp, 1e10, d2)

            colg = lax.broadcasted_iota(jnp.int32, d2.shape, 1) + base
            loc = []
            for _ in range(3):
                m = jnp.min(d2, axis=1, keepdims=True)
                cand = jnp.where(d2 == m, colg, 2 * N)
                j = jnp.min(cand, axis=1, keepdims=True)
                loc.append((m, j))
                d2 = jnp.where(cand == j, 3e10, d2)

            A = [(carry[k], carry[3 + k]) for k in range(3)]
            B = loc

            def lexlt(x, y):
                return (x[0] < y[0]) | ((x[0] == y[0]) & (x[1] < y[1]))

            def lmin(x, y):
                lt = lexlt(x, y)
                return (jnp.where(lt, x[0], y[0]), jnp.where(lt, x[1], y[1]))

            def lmax(x, y):
                lt = lexlt(x, y)
                return (jnp.where(lt, y[0], x[0]), jnp.where(lt, y[1], x[1]))

            m1 = lmin(A[0], B[0])
            m2 = lmin(lmin(lmax(A[0], B[0]), A[1]), B[1])
            m3 = lmin(lmin(A[2], lmax(A[1], B[0])),
                      lmin(lmax(A[0], B[1]), B[2]))
            return (m1[0], m2[0], m3[0], m1[1], m2[1], m3[1])

        init = (jnp.full((BLK, 1), 4e10, jnp.float32),) * 3 \
            + (jnp.full((BLK, 1), N, jnp.int32),) * 3
        c0 = c0_ref[g]
        res = lax.fori_loop(c0, c0 + nact_ref[g], chunk_body, init)

        part_ref[...] = (jnp.dot(xs_ref[...], w2_ref[...],
                                 preferred_element_type=jnp.float32)
                         + b_ref[0:1, :])
        ws = []
        den = jnp.zeros((BLK, 1), jnp.float32)
        for k in range(3):
            wk = 1.0 / jnp.maximum(res[k], 1e-16)
            ws.append(wk)
            den = den + wk
        inv_den = 1.0 / den
        idx_ref[...] = jnp.concatenate([res[3], res[4], res[5]], axis=1)
        wex_ref[...] = jnp.concatenate(
            [jnp.broadcast_to(w * inv_den, (BLK, 16)) for w in ws], axis=1)

    return body


_SC_CHUNK = 32


def _sc_combine(xp_hbm, idxf_hbm, wex_hbm, part_hbm, out_hbm,
                idx_v, w_v, g_v, o_v, in_sem, out_sem):
    nw = 32
    m_rows = out_hbm.shape[0]
    rows_per_w = m_rows // nw
    wid = lax.axis_index("s") * 2 + lax.axis_index("c")
    base = wid * rows_per_w
    chunk = _SC_CHUNK
    nchunks = rows_per_w // chunk
    fcols = out_hbm.shape[1]
    nvec = fcols // 16

    def fire(ci, slot):
        rbase = base + ci * chunk
        pltpu.sync_copy(idxf_hbm.at[pl.ds(rbase * 3, chunk * 3)], idx_v[slot])
        g = pltpu.async_copy(xp_hbm.at[idx_v[slot]], g_v[slot], in_sem[slot])
        w = pltpu.async_copy(wex_hbm.at[pl.ds(rbase, chunk)], w_v[slot],
                             in_sem[slot])
        p = pltpu.async_copy(part_hbm.at[pl.ds(rbase, chunk)], o_v[slot],
                             in_sem[slot])
        return (g, w, p)

    pending = {0: fire(0, 0)}
    out_pending = [None, None]

    for ci in range(nchunks):
        slot = ci % 2
        if ci + 1 < nchunks:
            nxt = (ci + 1) % 2
            if out_pending[nxt] is not None:
                out_pending[nxt].wait()
                out_pending[nxt] = None
            pending[nxt] = fire(ci + 1, nxt)
        for d in pending[slot]:
            d.wait()

        def row_body(r, _, slot=slot):
            w0 = w_v[slot][r, 0:16]
            w1 = w_v[slot][r, 16:32]
            w2 = w_v[slot][r, 32:48]
            for c in range(nvec):
                sl = pl.ds(c * 16, 16)
                acc = (g_v[slot][3 * r, sl] * w0 + g_v[slot][3 * r + 1, sl] * w1
                       + g_v[slot][3 * r + 2, sl] * w2 + o_v[slot][r, sl])
                o_v[slot][r, sl] = acc
            return 0

        lax.fori_loop(0, chunk, row_body, 0)
        rbase = base + ci * chunk
        out_pending[slot] = pltpu.async_copy(
            o_v[slot], out_hbm.at[pl.ds(rbase, chunk)], out_sem[slot])

    for s in (0, 1):
        if out_pending[s] is not None:
            out_pending[s].wait()


def kernel(x, pos, batch, x_skip, pos_skip, batch_skip, W, b):
    M = pos_skip.shape[0]
    N = pos.shape[0]
    F = x.shape[1]
    Fs = x_skip.shape[1]
    BLK = 256

    W1 = W[:F, :]
    W2 = W[F:, :]

    xp = pl.pallas_call(
        _proj_kernel,
        grid=(8,),
        in_specs=[pl.BlockSpec((N // 8, F), lambda i: (i, 0)),
                  pl.BlockSpec((F, F), lambda i: (0, 0))],
        out_specs=pl.BlockSpec((N // 8, F), lambda i: (i, 0)),
        out_shape=jax.ShapeDtypeStruct((N, F), jnp.float32),
    )(x, W1)

    q = jnp.concatenate(
        [pos_skip, batch_skip.astype(jnp.float32)[:, None],
         jnp.zeros((M, 4), jnp.float32)], axis=1)
    pt_arr = jnp.concatenate(
        [pos.T, batch.astype(jnp.float32)[None, :],
         jnp.zeros((4, N), jnp.float32)], axis=0)
    b_arr = jnp.zeros((8, F), jnp.float32).at[0].set(b)

    CH = 512
    nchunks = N // CH
    seg_start = jnp.searchsorted(batch, jnp.arange(16, dtype=batch.dtype),
                                 side="left")
    seg_end = jnp.searchsorted(batch, jnp.arange(16, dtype=batch.dtype),
                               side="right")
    bmin = batch_skip[::BLK]
    bmax = batch_skip[BLK - 1::BLK]
    wstart = seg_start[bmin]
    wend = seg_end[bmax]
    c0 = (wstart // CH).astype(jnp.int32)
    c1 = jnp.maximum((jnp.maximum(wend, 1) - 1) // CH, c0)
    nact = (c1 - c0 + 1).astype(jnp.int32)

    part, idx3, wex = pl.pallas_call(
        _make_select(N, CH, BLK),
        grid_spec=pltpu.PrefetchScalarGridSpec(
            num_scalar_prefetch=2,
            grid=(M // BLK,),
            in_specs=[
                pl.BlockSpec((BLK, 8), lambda g, *_: (g, 0)),
                pl.BlockSpec((8, N), lambda g, *_: (0, 0)),
                pl.BlockSpec((BLK, Fs), lambda g, *_: (g, 0)),
                pl.BlockSpec((Fs, F), lambda g, *_: (0, 0)),
                pl.BlockSpec((8, F), lambda g, *_: (0, 0)),
            ],
            out_specs=[
                pl.BlockSpec((BLK, F), lambda g, *_: (g, 0)),
                pl.BlockSpec((BLK, 3), lambda g, *_: (g, 0)),
                pl.BlockSpec((BLK, 48), lambda g, *_: (g, 0)),
            ],
        ),
        out_shape=[
            jax.ShapeDtypeStruct((M, F), jnp.float32),
            jax.ShapeDtypeStruct((M, 3), jnp.int32),
            jax.ShapeDtypeStruct((M, 48), jnp.float32),
        ],
    )(c0, nact, q, pt_arr, x_skip, W2, b_arr)

    out = _sc_call(xp, idx3.reshape(M * 3), wex, part, M, F)

    return (out, pos_skip, batch_skip)


def _sc_call(xp, idx_flat, wex, part, M, F):
    chunk = _SC_CHUNK
    sc_fn = functools.partial(
        pl.kernel,
        out_type=jax.ShapeDtypeStruct((M, F), jnp.float32),
        mesh=plsc.VectorSubcoreMesh(core_axis_name="c", subcore_axis_name="s"),
        scratch_types=[
            [pltpu.VMEM((chunk * 3,), jnp.int32)] * 2,
            [pltpu.VMEM((chunk, 48), jnp.float32)] * 2,
            [pltpu.VMEM((chunk * 3, F), jnp.float32)] * 2,
            [pltpu.VMEM((chunk, F), jnp.float32)] * 2,
            [pltpu.SemaphoreType.DMA] * 2,
            [pltpu.SemaphoreType.DMA] * 2,
        ],
    )(_sc_combine)
    return sc_fn(xp, idx_flat, wex, part)
